# baseline (speedup 1.0000x reference)
import functools

import jax
import jax.numpy as jnp
from jax import lax
from jax.experimental import pallas as pl
from jax.experimental.pallas import tpu as pltpu
from jax.experimental.pallas import tpu_sc as plsc

_N = 10000
_E = 320000
_D = 128
_NCORES = 2
_NSUB = 16
_NW = _NCORES * _NSUB
_EW = _E // _NW
_ROWS_PER_SUB = _N // _NSUB
_BN = 1000


def _make_sc_agg(dp: int, g: int, nbuf: int):
  kpt = _EW // g
  mesh = plsc.VectorSubcoreMesh(core_axis_name="c", subcore_axis_name="s")

  @functools.partial(
      pl.kernel,
      mesh=mesh,
      compiler_params=pltpu.CompilerParams(use_tc_tiling_on_sc=False),
      out_type=jax.ShapeDtypeStruct((_NCORES * _N, dp), jnp.float32),
      scratch_types=[
          pltpu.VMEM((kpt, g), jnp.int32),
          pltpu.VMEM((kpt, g), jnp.int32),
          pltpu.VMEM_SHARED((_N, dp), jnp.float32),
          [pltpu.VMEM((g, dp), jnp.float32) for _ in range(nbuf)],
          [pltpu.SemaphoreType.DMA for _ in range(nbuf)],
          [pltpu.SemaphoreType.DMA for _ in range(nbuf)],
      ],
  )
  def sc_agg(p_hbm, src_hbm, dst_hbm, zeros_hbm, out_hbm,
             src_v, dst_v, agg_sh, rows, sems_g, sems_s):
    c = lax.axis_index("c")
    s = lax.axis_index("s")
    w = c * _NSUB + s

    zbase = s * _ROWS_PER_SUB
    pltpu.sync_copy(zeros_hbm.at[pl.ds(zbase, _ROWS_PER_SUB)],
                    agg_sh.at[pl.ds(zbase, _ROWS_PER_SUB)])
    pltpu.sync_copy(src_hbm.at[pl.ds(w * kpt, kpt)], src_v)
    pltpu.sync_copy(dst_hbm.at[pl.ds(w * kpt, kpt)], dst_v)
    plsc.subcore_barrier()

    def body(q, carry):
      k0 = q * nbuf
      descs = []
      for j in range(nbuf):
        @pl.when(q > 0)
        def _(j=j):
          pltpu.make_async_copy(rows[j], agg_sh.at[dst_v.at[k0 + j]],
                                sems_s[j]).wait()
        descs.append(
            pltpu.async_copy(p_hbm.at[src_v.at[k0 + j]], rows[j], sems_g[j]))
      for j in range(nbuf):
        descs[j].wait()
        pltpu.async_copy(rows[j], agg_sh.at[dst_v.at[k0 + j]], sems_s[j],
                         add=True)
      return carry

    lax.fori_loop(0, kpt // nbuf, body, 0)
    for j in range(nbuf):
      pltpu.make_async_copy(rows[j], agg_sh.at[dst_v.at[j]],
                            sems_s[j]).wait()
    plsc.subcore_barrier()

    pltpu.sync_copy(agg_sh.at[pl.ds(zbase, _ROWS_PER_SUB)],
                    out_hbm.at[pl.ds(c * _N + zbase, _ROWS_PER_SUB)])

  return sc_agg


def _make_sc_agg_spmem(dp: int, g: int, nbuf: int):
  kpt = _EW // g
  mesh = plsc.VectorSubcoreMesh(core_axis_name="c", subcore_axis_name="s")

  @functools.partial(
      pl.kernel,
      mesh=mesh,
      compiler_params=pltpu.CompilerParams(use_tc_tiling_on_sc=False),
      out_type=jax.ShapeDtypeStruct((_NCORES * _N, dp), jnp.float32),
      scratch_types=[
          pltpu.VMEM((kpt, g), jnp.int32),
          pltpu.VMEM((kpt, g), jnp.int32),
          pltpu.VMEM_SHARED((_N, dp), jnp.float32),
          pltpu.VMEM_SHARED((_N, dp), jnp.float32),
          [pltpu.VMEM((g, dp), jnp.float32) for _ in range(nbuf)],
          [pltpu.SemaphoreType.DMA for _ in range(nbuf)],
          [pltpu.SemaphoreType.DMA for _ in range(nbuf)],
      ],
  )
  def sc_agg(p_hbm, src_hbm, dst_hbm, zeros_hbm, out_hbm,
             src_v, dst_v, p_sh, agg_sh, rows, sems_g, sems_s):
    c = lax.axis_index("c")
    s = lax.axis_index("s")
    w = c * _NSUB + s

    zbase = s * _ROWS_PER_SUB
    pltpu.sync_copy(p_hbm.at[pl.ds(zbase, _ROWS_PER_SUB)],
                    p_sh.at[pl.ds(zbase, _ROWS_PER_SUB)])
    pltpu.sync_copy(zeros_hbm.at[pl.ds(zbase, _ROWS_PER_SUB)],
                    agg_sh.at[pl.ds(zbase, _ROWS_PER_SUB)])
    pltpu.sync_copy(src_hbm.at[pl.ds(w * kpt, kpt)], src_v)
    pltpu.sync_copy(dst_hbm.at[pl.ds(w * kpt, kpt)], dst_v)
    plsc.subcore_barrier()

    def body(q, carry):
      k0 = q * nbuf
      descs = []
      for j in range(nbuf):
        @pl.when(q > 0)
        def _(j=j):
          pltpu.make_async_copy(rows[j], agg_sh.at[dst_v.at[k0 + j]],
                                sems_s[j]).wait()
        descs.append(
            pltpu.async_copy(p_sh.at[src_v.at[k0 + j]], rows[j], sems_g[j]))
      for j in range(nbuf):
        descs[j].wait()
        pltpu.async_copy(rows[j], agg_sh.at[dst_v.at[k0 + j]], sems_s[j],
                         add=True)
      return carry

    lax.fori_loop(0, kpt // nbuf, body, 0)
    for j in range(nbuf):
      pltpu.make_async_copy(rows[j], agg_sh.at[dst_v.at[j]],
                            sems_s[j]).wait()
    plsc.subcore_barrier()

    pltpu.sync_copy(agg_sh.at[pl.ds(zbase, _ROWS_PER_SUB)],
                    out_hbm.at[pl.ds(c * _N + zbase, _ROWS_PER_SUB)])

  return sc_agg

_G128, _NBUF128 = 40, 5
_G64, _NBUF64 = 50, 8
_sc_agg_128 = _make_sc_agg(_D, _G128, _NBUF128)
_sc_agg_64 = _make_sc_agg_spmem(64, _G64, _NBUF64)


def _tc_first(x, w_self, w_neigh, b):
  def body(x_ref, ws_ref, wn_ref, b_ref, s_ref, p_ref):
    xb = x_ref[...]
    s_ref[...] = jnp.dot(xb, ws_ref[...],
                         preferred_element_type=jnp.float32) + b_ref[...]
    p_ref[...] = jnp.dot(xb, wn_ref[...], preferred_element_type=jnp.float32)

  return pl.pallas_call(
      body,
      grid=(_N // _BN,),
      in_specs=[
          pl.BlockSpec((_BN, _D), lambda i: (i, 0)),
          pl.BlockSpec((_D, _D), lambda i: (0, 0)),
          pl.BlockSpec((_D, _D), lambda i: (0, 0)),
          pl.BlockSpec((1, _D), lambda i: (0, 0)),
      ],
      out_specs=[
          pl.BlockSpec((_BN, _D), lambda i: (i, 0)),
          pl.BlockSpec((_BN, _D), lambda i: (i, 0)),
      ],
      out_shape=[jax.ShapeDtypeStruct((_N, _D), jnp.float32)] * 2,
  )(x, w_self, w_neigh, b.reshape(1, _D))


def _tc_combine_next(s_prev, parts, deg, w_self, w_neigh, b, dp_in, dp_out):
  def body(s_ref, a0_ref, a1_ref, deg_ref, ws_ref, wn_ref, b_ref,
           s_out, p_out):
    h = s_ref[...] + (a0_ref[...] + a1_ref[...]) / deg_ref[...]
    h = jnp.maximum(h, 0.0)
    s_out[...] = jnp.dot(h, ws_ref[...],
                         preferred_element_type=jnp.float32) + b_ref[...]
    p_out[...] = jnp.dot(h, wn_ref[...], preferred_element_type=jnp.float32)

  nblk = _N // _BN
  return pl.pallas_call(
      body,
      grid=(nblk,),
      in_specs=[
          pl.BlockSpec((_BN, dp_in), lambda i: (i, 0)),
          pl.BlockSpec((_BN, dp_in), lambda i: (i, 0)),
          pl.BlockSpec((_BN, dp_in), lambda i: (i + nblk, 0)),
          pl.BlockSpec((_BN, 1), lambda i: (i, 0)),
          pl.BlockSpec((dp_in, dp_out), lambda i: (0, 0)),
          pl.BlockSpec((dp_in, dp_out), lambda i: (0, 0)),
          pl.BlockSpec((1, dp_out), lambda i: (0, 0)),
      ],
      out_specs=[
          pl.BlockSpec((_BN, dp_out), lambda i: (i, 0)),
          pl.BlockSpec((_BN, dp_out), lambda i: (i, 0)),
      ],
      out_shape=[jax.ShapeDtypeStruct((_N, dp_out), jnp.float32)] * 2,
  )(s_prev, parts, parts, deg, w_self, w_neigh, b.reshape(1, dp_out))


def _tc_final(s_prev, parts, deg, dp):
  def body(s_ref, a0_ref, a1_ref, deg_ref, o_ref):
    o_ref[...] = s_ref[...] + (a0_ref[...] + a1_ref[...]) / deg_ref[...]

  nblk = _N // _BN
  return pl.pallas_call(
      body,
      grid=(nblk,),
      in_specs=[
          pl.BlockSpec((_BN, dp), lambda i: (i, 0)),
          pl.BlockSpec((_BN, dp), lambda i: (i, 0)),
          pl.BlockSpec((_BN, dp), lambda i: (i + nblk, 0)),
          pl.BlockSpec((_BN, 1), lambda i: (i, 0)),
      ],
      out_specs=pl.BlockSpec((_BN, dp), lambda i: (i, 0)),
      out_shape=jax.ShapeDtypeStruct((_N, dp), jnp.float32),
  )(s_prev, parts, parts, deg)


def kernel(x, edge_index, in_degrees,
           W_self0, W_neigh0, b0,
           W_self1, W_neigh1, b1,
           W_self2, W_neigh2, b2):
  src128 = edge_index[0].reshape(_E // _G128, _G128)
  dst128 = edge_index[1].reshape(_E // _G128, _G128)
  src64 = edge_index[0].reshape(_E // _G64, _G64)
  dst64 = edge_index[1].reshape(_E // _G64, _G64)
  deg = in_degrees.reshape(_N, 1)
  zeros128 = jnp.zeros((_N, _D), jnp.float32)
  zeros64 = jnp.zeros((_N, 64), jnp.float32)

  w_self2p = jnp.pad(W_self2, ((0, 0), (0, 64 - 47)))
  w_neigh2p = jnp.pad(W_neigh2, ((0, 0), (0, 64 - 47)))
  b2p = jnp.pad(b2, (0, 64 - 47))

  s0, p0 = _tc_first(x, W_self0, W_neigh0, b0)
  a0 = _sc_agg_128(p0, src128, dst128, zeros128)
  s1, p1 = _tc_combine_next(s0, a0, deg, W_self1, W_neigh1, b1, _D, _D)
  a1 = _sc_agg_128(p1, src128, dst128, zeros128)
  s2, p2 = _tc_combine_next(s1, a1, deg, w_self2p, w_neigh2p, b2p, _D, 64)
  a2 = _sc_agg_64(p2, src64, dst64, zeros64)
  out = _tc_final(s2, a2, deg, 64)
  return out[:, :47]


# speedup vs baseline: 1.0889x; 1.0889x over previous
import functools

import jax
import jax.numpy as jnp
from jax import lax
from jax.experimental import pallas as pl
from jax.experimental.pallas import tpu as pltpu
from jax.experimental.pallas import tpu_sc as plsc

_N = 10000
_E = 320000
_D = 128
_NCORES = 2
_NSUB = 16
_NW = _NCORES * _NSUB
_EW = _E // _NW
_ROWS_PER_SUB = _N // _NSUB
_BN = 1000


def _make_sc_agg(dp: int, g: int, nbuf: int):
  kpt = _EW // g
  mesh = plsc.VectorSubcoreMesh(core_axis_name="c", subcore_axis_name="s")

  @functools.partial(
      pl.kernel,
      mesh=mesh,
      compiler_params=pltpu.CompilerParams(use_tc_tiling_on_sc=False),
      out_type=jax.ShapeDtypeStruct((_NCORES * _N, dp), jnp.float32),
      scratch_types=[
          pltpu.VMEM((kpt, g), jnp.int32),
          pltpu.VMEM((kpt, g), jnp.int32),
          pltpu.VMEM_SHARED((_N, dp), jnp.float32),
          [pltpu.VMEM((g, dp), jnp.float32) for _ in range(nbuf)],
          [pltpu.SemaphoreType.DMA for _ in range(nbuf)],
          [pltpu.SemaphoreType.DMA for _ in range(nbuf)],
      ],
  )
  def sc_agg(p_hbm, src_hbm, dst_hbm, zeros_hbm, out_hbm,
             src_v, dst_v, agg_sh, rows, sems_g, sems_s):
    c = lax.axis_index("c")
    s = lax.axis_index("s")
    w = c * _NSUB + s

    zbase = s * _ROWS_PER_SUB
    d0 = pltpu.async_copy(zeros_hbm.at[pl.ds(zbase, _ROWS_PER_SUB)],
                          agg_sh.at[pl.ds(zbase, _ROWS_PER_SUB)], sems_g[0])
    d1 = pltpu.async_copy(src_hbm.at[pl.ds(w * kpt, kpt)], src_v, sems_g[1])
    d2 = pltpu.async_copy(dst_hbm.at[pl.ds(w * kpt, kpt)], dst_v, sems_g[2])
    d0.wait()
    d1.wait()
    d2.wait()
    plsc.subcore_barrier()

    def body(q, carry):
      k0 = q * nbuf
      descs = []
      for j in range(nbuf):
        @pl.when(q > 0)
        def _(j=j):
          pltpu.make_async_copy(rows[j], agg_sh.at[dst_v.at[k0 + j]],
                                sems_s[j]).wait()
        descs.append(
            pltpu.async_copy(p_hbm.at[src_v.at[k0 + j]], rows[j], sems_g[j]))
      for j in range(nbuf):
        descs[j].wait()
        pltpu.async_copy(rows[j], agg_sh.at[dst_v.at[k0 + j]], sems_s[j],
                         add=True)
      return carry

    lax.fori_loop(0, kpt // nbuf, body, 0)
    for j in range(nbuf):
      pltpu.make_async_copy(rows[j], agg_sh.at[dst_v.at[j]],
                            sems_s[j]).wait()
    plsc.subcore_barrier()

    pltpu.sync_copy(agg_sh.at[pl.ds(zbase, _ROWS_PER_SUB)],
                    out_hbm.at[pl.ds(c * _N + zbase, _ROWS_PER_SUB)])

  return sc_agg


_G128, _NBUF128 = 40, 5
_G64, _NBUF64 = 100, 5
_sc_agg_128 = _make_sc_agg(_D, _G128, _NBUF128)
_sc_agg_64 = _make_sc_agg(64, _G64, _NBUF64)


def _tc_first(x, w_self, w_neigh, b):
  def body(x_ref, ws_ref, wn_ref, b_ref, s_ref, p_ref):
    xb = x_ref[...]
    s_ref[...] = jnp.dot(xb, ws_ref[...],
                         preferred_element_type=jnp.float32) + b_ref[...]
    p_ref[...] = jnp.dot(xb, wn_ref[...], preferred_element_type=jnp.float32)

  return pl.pallas_call(
      body,
      grid=(_N // _BN,),
      in_specs=[
          pl.BlockSpec((_BN, _D), lambda i: (i, 0)),
          pl.BlockSpec((_D, _D), lambda i: (0, 0)),
          pl.BlockSpec((_D, _D), lambda i: (0, 0)),
          pl.BlockSpec((1, _D), lambda i: (0, 0)),
      ],
      out_specs=[
          pl.BlockSpec((_BN, _D), lambda i: (i, 0)),
          pl.BlockSpec((_BN, _D), lambda i: (i, 0)),
      ],
      out_shape=[jax.ShapeDtypeStruct((_N, _D), jnp.float32)] * 2,
  )(x, w_self, w_neigh, b.reshape(1, _D))


def _tc_combine_next(s_prev, parts, deg, w_self, w_neigh, b, dp_in, dp_out):
  def body(s_ref, a0_ref, a1_ref, deg_ref, ws_ref, wn_ref, b_ref,
           s_out, p_out):
    h = s_ref[...] + (a0_ref[...] + a1_ref[...]) / deg_ref[...]
    h = jnp.maximum(h, 0.0)
    s_out[...] = jnp.dot(h, ws_ref[...],
                         preferred_element_type=jnp.float32) + b_ref[...]
    p_out[...] = jnp.dot(h, wn_ref[...], preferred_element_type=jnp.float32)

  nblk = _N // _BN
  return pl.pallas_call(
      body,
      grid=(nblk,),
      in_specs=[
          pl.BlockSpec((_BN, dp_in), lambda i: (i, 0)),
          pl.BlockSpec((_BN, dp_in), lambda i: (i, 0)),
          pl.BlockSpec((_BN, dp_in), lambda i: (i + nblk, 0)),
          pl.BlockSpec((_BN, 1), lambda i: (i, 0)),
          pl.BlockSpec((dp_in, dp_out), lambda i: (0, 0)),
          pl.BlockSpec((dp_in, dp_out), lambda i: (0, 0)),
          pl.BlockSpec((1, dp_out), lambda i: (0, 0)),
      ],
      out_specs=[
          pl.BlockSpec((_BN, dp_out), lambda i: (i, 0)),
          pl.BlockSpec((_BN, dp_out), lambda i: (i, 0)),
      ],
      out_shape=[jax.ShapeDtypeStruct((_N, dp_out), jnp.float32)] * 2,
  )(s_prev, parts, parts, deg, w_self, w_neigh, b.reshape(1, dp_out))


def _tc_final(s_prev, parts, deg, dp):
  def body(s_ref, a0_ref, a1_ref, deg_ref, o_ref):
    o_ref[...] = (s_ref[...] + (a0_ref[...] + a1_ref[...]) / deg_ref[...])[:, :47]

  nblk = _N // _BN
  return pl.pallas_call(
      body,
      grid=(nblk,),
      in_specs=[
          pl.BlockSpec((_BN, dp), lambda i: (i, 0)),
          pl.BlockSpec((_BN, dp), lambda i: (i, 0)),
          pl.BlockSpec((_BN, dp), lambda i: (i + nblk, 0)),
          pl.BlockSpec((_BN, 1), lambda i: (i, 0)),
      ],
      out_specs=pl.BlockSpec((_BN, 47), lambda i: (i, 0)),
      out_shape=jax.ShapeDtypeStruct((_N, 47), jnp.float32),
  )(s_prev, parts, parts, deg)


def kernel(x, edge_index, in_degrees,
           W_self0, W_neigh0, b0,
           W_self1, W_neigh1, b1,
           W_self2, W_neigh2, b2):
  src128 = edge_index[0].reshape(_E // _G128, _G128)
  dst128 = edge_index[1].reshape(_E // _G128, _G128)
  src64 = edge_index[0].reshape(_E // _G64, _G64)
  dst64 = edge_index[1].reshape(_E // _G64, _G64)
  deg = in_degrees.reshape(_N, 1)
  zeros128 = jnp.zeros((_N, _D), jnp.float32)
  zeros64 = jnp.zeros((_N, 64), jnp.float32)

  w_self2p = jnp.pad(W_self2, ((0, 0), (0, 64 - 47)))
  w_neigh2p = jnp.pad(W_neigh2, ((0, 0), (0, 64 - 47)))
  b2p = jnp.pad(b2, (0, 64 - 47))

  s0, p0 = _tc_first(x, W_self0, W_neigh0, b0)
  a0 = _sc_agg_128(p0, src128, dst128, zeros128)
  s1, p1 = _tc_combine_next(s0, a0, deg, W_self1, W_neigh1, b1, _D, _D)
  a1 = _sc_agg_128(p1, src128, dst128, zeros128)
  s2, p2 = _tc_combine_next(s1, a1, deg, w_self2p, w_neigh2p, b2p, _D, 64)
  a2 = _sc_agg_64(p2, src64, dst64, zeros64)
  return _tc_final(s2, a2, deg, 64)
